# Initial kernel scaffold; baseline (speedup 1.0000x reference)
import jax
import jax.numpy as jnp
from jax.experimental import pallas as pl


def kernel(input_ids, attention_mask, position_ids, keys, db_chunk_ids, query_table, embed_table, ln_gamma, ln_beta, Wq, Wk, Wv, Wo):
    raise NotImplementedError("write your pallas kernel here")


# speedup vs baseline: 1.5039x; 1.5039x over previous
import functools

import jax
import jax.numpy as jnp
from jax import lax
from jax.experimental import pallas as pl
from jax.experimental.pallas import tpu as pltpu
from jax.experimental.pallas import tpu_sc as plsc

VOCAB = 32000
HIDDEN = 1024
HEADS = 16
CHUNK = 32
NKEYS = 100000
KDIM = 256
DH = HIDDEN // HEADS

_SC_MESH = plsc.VectorSubcoreMesh(core_axis_name="c", subcore_axis_name="s")


@functools.partial(
    pl.kernel,
    out_type=jax.ShapeDtypeStruct((KDIM,), jnp.float32),
    mesh=_SC_MESH,
    scratch_types=[
        pltpu.VMEM((CHUNK,), jnp.int32),
        pltpu.VMEM((CHUNK, KDIM), jnp.float32),
        pltpu.VMEM((KDIM,), jnp.float32),
        pltpu.SemaphoreType.DMA,
    ],
)
def _sc_qemb(ids_hbm, qtab_hbm, out_hbm, idx_v, rows_v, qemb_v, sem):
    cid = lax.axis_index("c")
    sid = lax.axis_index("s")

    @pl.when(jnp.logical_and(cid == 0, sid == 0))
    def _():
        pltpu.sync_copy(ids_hbm.at[pl.ds(CHUNK, CHUNK)], idx_v)
        pltpu.async_copy(qtab_hbm.at[idx_v], rows_v, sem).wait()
        inv = 1.0 / float(CHUNK)
        for dc in range(KDIM // 16):
            acc = rows_v[0, pl.ds(dc * 16, 16)]
            for r in range(1, CHUNK):
                acc = acc + rows_v[r, pl.ds(dc * 16, 16)]
            qemb_v[pl.ds(dc * 16, 16)] = acc * inv
        pltpu.sync_copy(qemb_v, out_hbm)


_BLK = 2000
_NB = NKEYS // _BLK


def _tc_argmax_body(qe_ref, keys_ref, out_ref, best_v, best_i):
    i = pl.program_id(0)

    @pl.when(i == 0)
    def _():
        best_v[0] = -jnp.inf
        best_i[0] = 0

    s = lax.dot_general(
        keys_ref[...], qe_ref[...],
        (((1,), (1,)), ((), ())),
        preferred_element_type=jnp.float32,
    )
    m = jnp.max(s)
    iota0 = lax.broadcasted_iota(jnp.int32, (_BLK, 1), 0)
    li = jnp.min(jnp.where(s >= m, iota0, NKEYS))

    @pl.when(m > best_v[0])
    def _():
        best_v[0] = m
        best_i[0] = i * _BLK + li

    @pl.when(i == _NB - 1)
    def _():
        for j in range(16):
            out_ref[j] = best_i[0]


def _tc_argmax(q_emb, keys):
    return pl.pallas_call(
        _tc_argmax_body,
        grid=(_NB,),
        in_specs=[
            pl.BlockSpec((1, KDIM), lambda i: (0, 0)),
            pl.BlockSpec((_BLK, KDIM), lambda i: (i, 0)),
        ],
        out_specs=pl.BlockSpec(memory_space=pltpu.SMEM),
        out_shape=jax.ShapeDtypeStruct((16,), jnp.int32),
        scratch_shapes=[
            pltpu.SMEM((1,), jnp.float32),
            pltpu.SMEM((1,), jnp.int32),
        ],
        compiler_params=pltpu.CompilerParams(
            dimension_semantics=("arbitrary",),
        ),
    )(q_emb, keys)


@functools.partial(
    pl.kernel,
    out_type=jax.ShapeDtypeStruct((CHUNK, HIDDEN), jnp.float32),
    mesh=_SC_MESH,
    scratch_types=[
        pltpu.VMEM((16,), jnp.int32),
        pltpu.VMEM((CHUNK,), jnp.int32),
        pltpu.VMEM((CHUNK, HIDDEN), jnp.float32),
        pltpu.SemaphoreType.DMA,
    ],
)
def _sc_retrieve(tidx_hbm, db_hbm, emb_hbm, out_hbm,
                 tidx_v, ids_v, rows_v, sem1):
    cid = lax.axis_index("c")
    sid = lax.axis_index("s")

    @pl.when(jnp.logical_and(cid == 0, sid == 0))
    def _():
        pltpu.sync_copy(tidx_hbm, tidx_v)
        t0 = tidx_v[...][0]
        pltpu.sync_copy(db_hbm.at[pl.ds(t0 * CHUNK, CHUNK)], ids_v)
        pltpu.async_copy(emb_hbm.at[ids_v], rows_v, sem1).wait()
        pltpu.sync_copy(rows_v, out_hbm)


def _tc_attn_body(h_ref, mask_ref, pos_ref, g_ref, b_ref,
                  wq_ref, wk_ref, wv_ref, wo_ref, out_ref, o_scr):
    h = h_ref[...]
    mu = jnp.mean(h, axis=1, keepdims=True)
    xc = h - mu
    var = jnp.mean(xc * xc, axis=1, keepdims=True)
    hn = xc * lax.rsqrt(var + 1e-5) * g_ref[...] + b_ref[...]

    dot = functools.partial(
        jnp.dot, preferred_element_type=jnp.float32,
        precision=lax.Precision.HIGHEST)
    q = dot(hn, wq_ref[...])
    k = dot(hn, wk_ref[...])
    v = dot(hn, wv_ref[...])

    jj = lax.broadcasted_iota(jnp.int32, (1, DH // 2), 1).astype(jnp.float32)
    inv_freq = jnp.exp(jj * (-2.0 / DH * 9.210340371976184))
    ang = pos_ref[...] * inv_freq
    cos_t = jnp.cos(ang)
    sin_t = jnp.sin(ang)

    def rope(x):
        x1 = x[:, : DH // 2]
        x2 = x[:, DH // 2:]
        return jnp.concatenate(
            [x1 * cos_t - x2 * sin_t, x2 * cos_t + x1 * sin_t], axis=1)

    scale = 1.0 / (DH ** 0.5)
    mask = mask_ref[...]
    for hd in range(HEADS):
        sl = slice(hd * DH, (hd + 1) * DH)
        qh = rope(q[:, sl])
        kh = rope(k[:, sl])
        att = lax.dot_general(
            qh, kh, (((1,), (1,)), ((), ())),
            preferred_element_type=jnp.float32,
            precision=lax.Precision.HIGHEST) * scale + mask
        att = att - jnp.max(att, axis=1, keepdims=True)
        e = jnp.exp(att)
        p = e / jnp.sum(e, axis=1, keepdims=True)
        o_scr[:, sl] = dot(p, v[:, sl])
    out_ref[...] = dot(o_scr[...], wo_ref[...])


def _tc_attn(h, mask2d, posf, gamma, beta, Wq, Wk, Wv, Wo):
    return pl.pallas_call(
        _tc_attn_body,
        out_shape=jax.ShapeDtypeStruct((CHUNK, HIDDEN), jnp.float32),
        scratch_shapes=[pltpu.VMEM((CHUNK, HIDDEN), jnp.float32)],
    )(h, mask2d, posf, gamma, beta, Wq, Wk, Wv, Wo)


def kernel(input_ids, attention_mask, position_ids, keys, db_chunk_ids,
           query_table, embed_table, ln_gamma, ln_beta, Wq, Wk, Wv, Wo):
    ids_flat = input_ids.reshape(-1).astype(jnp.int32)
    q_emb = _sc_qemb(ids_flat, query_table)
    top_idx = _tc_argmax(q_emb.reshape(1, KDIM), keys)
    h = _sc_retrieve(top_idx, db_chunk_ids.reshape(-1), embed_table)
    mask2d = attention_mask.reshape(CHUNK, CHUNK)
    posf = position_ids.reshape(CHUNK, 1).astype(jnp.float32)
    out = _tc_attn(h, mask2d, posf,
                   ln_gamma.reshape(1, HIDDEN), ln_beta.reshape(1, HIDDEN),
                   Wq, Wk, Wv, Wo)
    return out.reshape(1, CHUNK, HIDDEN)
